# baseline (speedup 1.0000x reference)
import functools

import jax
import jax.numpy as jnp
from jax import lax
from jax.experimental import pallas as pl
from jax.experimental.pallas import tpu as pltpu
from jax.experimental.pallas import tpu_sc as plsc

B = 16
MAX_LEN = 2048
D = 768
NW = 32
NS = 16
CHUNK = MAX_LEN // NW
L = 16


TILE_SRC = (1, 4, 7, 10, 12, 15)


def _pos_body(table_hbm, len_hbm, out_hbm,
              len_v, idx_v, idx_z, tbuf, zbuf, sh_t, sh_z,
              sem_g, sem_ws, sem_wt):
    cid = lax.axis_index("c")
    sid = lax.axis_index("s")
    wid = sid * 2 + cid
    s = wid * CHUNK

    pltpu.sync_copy(len_hbm, len_v)
    lens = len_v[...]
    iota = lax.iota(jnp.int32, L)

    for j in range(CHUNK // (2 * L)):
        idx_z[pl.ds(j * L, L)] = jnp.zeros((L,), jnp.int32)
    pltpu.async_copy(table_hbm.at[idx_z], zbuf, sem_g).wait()

    @pl.when(sid == 0)
    def _():
        pltpu.sync_copy(zbuf, sh_z.at[pl.ds(0, CHUNK // 2)])
        pltpu.sync_copy(zbuf, sh_z.at[pl.ds(CHUNK // 2, CHUNK // 2)])

    for j in range(CHUNK // L):
        idx_v[pl.ds(j * L, L)] = s + j * L + iota + 1
    pltpu.async_copy(table_hbm.at[idx_v], tbuf, sem_g).wait()
    pltpu.sync_copy(tbuf, sh_t.at[sid])

    plsc.subcore_barrier()

    n_sp = jnp.int32(0)
    n_tl = jnp.int32(0)
    for b in range(B):
        lb = lens[b]
        outside = (s + CHUNK <= lb) | (lb <= s)

        if b in TILE_SRC:
            @pl.when(s + CHUNK <= lb)
            def _():
                pltpu.async_copy(tbuf, out_hbm.at[b, pl.ds(s, CHUNK)], sem_wt)

            @pl.when(lb <= s)
            def _():
                pltpu.async_copy(zbuf, out_hbm.at[b, pl.ds(s, CHUNK // 2)],
                                 sem_wt)
                pltpu.async_copy(zbuf,
                                 out_hbm.at[b, pl.ds(s + CHUNK // 2,
                                                     CHUNK // 2)], sem_wt)

            n_tl = n_tl + jnp.where(outside, 2, 0).astype(jnp.int32)
        else:
            @pl.when(s + CHUNK <= lb)
            def _():
                pltpu.async_copy(sh_t.at[sid], out_hbm.at[b, pl.ds(s, CHUNK)],
                                 sem_ws)

            @pl.when(lb <= s)
            def _():
                pltpu.async_copy(sh_z, out_hbm.at[b, pl.ds(s, CHUNK)], sem_ws)

            n_sp = n_sp + jnp.where(outside, 1, 0).astype(jnp.int32)

    def drain_tl(i, carry):
        @pl.when(i < n_tl)
        def _():
            pltpu.make_async_copy(zbuf, out_hbm.at[0, pl.ds(0, CHUNK // 2)],
                                  sem_wt).wait()
        return carry

    lax.fori_loop(0, 2 * B, drain_tl, 0)

    def drain_sp(i, carry):
        @pl.when(i < n_sp)
        def _():
            pltpu.make_async_copy(sh_z, out_hbm.at[0, pl.ds(0, CHUNK)],
                                  sem_ws).wait()
        return carry

    lax.fori_loop(0, B, drain_sp, 0)

    for b in range(B):
        lb = lens[b]

        @pl.when((s < lb) & (lb < s + CHUNK))
        def _():
            for j in range(CHUNK // L):
                vec = s + j * L + iota + 1
                idx_v[pl.ds(j * L, L)] = jnp.where(vec <= lb, vec, 0)
            pltpu.async_copy(table_hbm.at[idx_v], tbuf, sem_g).wait()
            pltpu.sync_copy(tbuf, out_hbm.at[b, pl.ds(s, CHUNK)])


def kernel(input_len, table):
    len_i32 = input_len.astype(jnp.int32)
    mesh = plsc.VectorSubcoreMesh(core_axis_name="c", subcore_axis_name="s")
    run = functools.partial(
        pl.kernel,
        mesh=mesh,
        out_type=jax.ShapeDtypeStruct((B, MAX_LEN, D), jnp.float32),
        scratch_types=[
            pltpu.VMEM((L,), jnp.int32),
            pltpu.VMEM((CHUNK,), jnp.int32),
            pltpu.VMEM((CHUNK // 2,), jnp.int32),
            pltpu.VMEM((CHUNK, D), jnp.float32),
            pltpu.VMEM((CHUNK // 2, D), jnp.float32),
            pltpu.VMEM_SHARED((NS, CHUNK, D), jnp.float32),
            pltpu.VMEM_SHARED((CHUNK, D), jnp.float32),
            pltpu.SemaphoreType.DMA,
            pltpu.SemaphoreType.DMA,
            pltpu.SemaphoreType.DMA,
        ],
    )(_pos_body)
    return run(table, len_i32)


# speedup vs baseline: 1.1783x; 1.1783x over previous
import functools

import jax
import jax.numpy as jnp
from jax import lax
from jax.experimental import pallas as pl
from jax.experimental.pallas import tpu as pltpu
from jax.experimental.pallas import tpu_sc as plsc

B = 16
MAX_LEN = 2048
D = 768
NW = 32
NS = 16
CHUNK = MAX_LEN // NW
L = 16


def _pos_body(table_hbm, tshift_hbm, len_hbm, out_hbm,
              len_v, idx_v, tbuf, sh_t, sh_z, sem_g, sem_w):
    cid = lax.axis_index("c")
    sid = lax.axis_index("s")
    wid = sid * 2 + cid
    s = wid * CHUNK

    pltpu.sync_copy(len_hbm, len_v)
    lens = len_v[...]
    iota = lax.iota(jnp.int32, L)

    @pl.when(sid == 0)
    def _():
        for j in range(CHUNK // L):
            idx_v[pl.ds(j * L, L)] = jnp.zeros((L,), jnp.int32)
        pltpu.async_copy(table_hbm.at[idx_v], tbuf, sem_g).wait()
        pltpu.sync_copy(tbuf, sh_z)

    pltpu.sync_copy(tshift_hbm.at[pl.ds(s, CHUNK)], sh_t.at[sid])

    plsc.subcore_barrier()

    n_async = jnp.int32(0)
    for b in range(B):
        lb = lens[b]

        @pl.when(s + CHUNK <= lb)
        def _():
            pltpu.async_copy(sh_t.at[sid], out_hbm.at[b, pl.ds(s, CHUNK)],
                             sem_w)

        @pl.when(lb <= s)
        def _():
            pltpu.async_copy(sh_z, out_hbm.at[b, pl.ds(s, CHUNK)], sem_w)

        outside = (s + CHUNK <= lb) | (lb <= s)
        n_async = n_async + jnp.where(outside, 1, 0).astype(jnp.int32)

    def drain(i, carry):
        @pl.when(i < n_async)
        def _():
            pltpu.make_async_copy(sh_z, out_hbm.at[0, pl.ds(0, CHUNK)],
                                  sem_w).wait()
        return carry

    lax.fori_loop(0, B, drain, 0)

    for b in range(B):
        lb = lens[b]

        @pl.when((s < lb) & (lb < s + CHUNK))
        def _():
            for j in range(CHUNK // L):
                vec = s + j * L + iota + 1
                idx_v[pl.ds(j * L, L)] = jnp.where(vec <= lb, vec, 0)
            pltpu.async_copy(table_hbm.at[idx_v], tbuf, sem_g).wait()
            pltpu.sync_copy(tbuf, out_hbm.at[b, pl.ds(s, CHUNK)])


def kernel(input_len, table):
    len_i32 = input_len.astype(jnp.int32)
    mesh = plsc.VectorSubcoreMesh(core_axis_name="c", subcore_axis_name="s")
    run = functools.partial(
        pl.kernel,
        mesh=mesh,
        out_type=jax.ShapeDtypeStruct((B, MAX_LEN, D), jnp.float32),
        scratch_types=[
            pltpu.VMEM((L,), jnp.int32),
            pltpu.VMEM((CHUNK,), jnp.int32),
            pltpu.VMEM((CHUNK, D), jnp.float32),
            pltpu.VMEM_SHARED((NS, CHUNK, D), jnp.float32),
            pltpu.VMEM_SHARED((CHUNK, D), jnp.float32),
            pltpu.SemaphoreType.DMA,
            pltpu.SemaphoreType.DMA,
        ],
    )(_pos_body)
    tshift = lax.slice(table, (1, 0), (MAX_LEN + 1, D))
    return run(table, tshift, len_i32)


# speedup vs baseline: 1.4028x; 1.1906x over previous
import functools

import jax
import jax.numpy as jnp
from jax import lax
from jax.experimental import pallas as pl
from jax.experimental.pallas import tpu as pltpu
from jax.experimental.pallas import tpu_sc as plsc

B = 16
MAX_LEN = 2048
D = 768
K = 1536
NW = 32
NS = 16
CHUNK = (MAX_LEN - K) // NW
L = 16
BR = 256


def _sc_body(table_hbm, tshift_hbm, len_hbm, out_hbm,
             len_v, idx_v, tbuf, sh_t, sh_z, sem_g, sem_w):
    cid = lax.axis_index("c")
    sid = lax.axis_index("s")
    wid = sid * 2 + cid
    s = K + wid * CHUNK

    pltpu.sync_copy(len_hbm, len_v)
    lens = len_v[...]
    iota = lax.iota(jnp.int32, L)

    @pl.when(sid == 0)
    def _():
        for j in range(CHUNK // L):
            idx_v[pl.ds(j * L, L)] = jnp.zeros((L,), jnp.int32)
        pltpu.async_copy(table_hbm.at[idx_v], tbuf, sem_g).wait()
        pltpu.sync_copy(tbuf, sh_z)

    pltpu.sync_copy(tshift_hbm.at[pl.ds(s, CHUNK)], sh_t.at[sid])

    plsc.subcore_barrier()

    n_async = jnp.int32(0)
    for b in range(B):
        lb = lens[b]

        @pl.when(s + CHUNK <= lb)
        def _():
            pltpu.async_copy(sh_t.at[sid], out_hbm.at[b, pl.ds(s, CHUNK)],
                             sem_w)

        @pl.when(lb <= s)
        def _():
            pltpu.async_copy(sh_z, out_hbm.at[b, pl.ds(s, CHUNK)], sem_w)

        outside = (s + CHUNK <= lb) | (lb <= s)
        n_async = n_async + jnp.where(outside, 1, 0).astype(jnp.int32)

    def drain(i, carry):
        @pl.when(i < n_async)
        def _():
            pltpu.make_async_copy(sh_z, out_hbm.at[0, pl.ds(0, CHUNK)],
                                  sem_w).wait()
        return carry

    lax.fori_loop(0, B, drain, 0)

    for b in range(B):
        lb = lens[b]

        @pl.when((s < lb) & (lb < s + CHUNK))
        def _():
            for j in range(CHUNK // L):
                vec = s + j * L + iota + 1
                idx_v[pl.ds(j * L, L)] = jnp.where(vec <= lb, vec, 0)
            pltpu.async_copy(table_hbm.at[idx_v], tbuf, sem_g).wait()
            pltpu.sync_copy(tbuf, out_hbm.at[b, pl.ds(s, CHUNK)])


def _tc_body(_, tab_ref, len_ref, out_ref):
    i = pl.program_id(0)
    b = pl.program_id(1)
    lb = len_ref[b]
    rows = i * BR + lax.broadcasted_iota(jnp.int32, (BR, 1), 0)
    out_ref[...] = jnp.where(rows < lb, tab_ref[...], 0.0)[None]


def kernel(input_len, table):
    len_i32 = input_len.astype(jnp.int32)
    tshift = lax.slice(table, (1, 0), (MAX_LEN + 1, D))

    mesh = plsc.VectorSubcoreMesh(core_axis_name="c", subcore_axis_name="s")
    sc_run = functools.partial(
        pl.kernel,
        mesh=mesh,
        out_type=jax.ShapeDtypeStruct((B, MAX_LEN, D), jnp.float32),
        scratch_types=[
            pltpu.VMEM((L,), jnp.int32),
            pltpu.VMEM((CHUNK,), jnp.int32),
            pltpu.VMEM((CHUNK, D), jnp.float32),
            pltpu.VMEM_SHARED((NS, CHUNK, D), jnp.float32),
            pltpu.VMEM_SHARED((CHUNK, D), jnp.float32),
            pltpu.SemaphoreType.DMA,
            pltpu.SemaphoreType.DMA,
        ],
    )(_sc_body)
    out_sc = sc_run(table, tshift, len_i32)

    tc_run = pl.pallas_call(
        _tc_body,
        grid=(K // BR, B),
        in_specs=[
            pl.BlockSpec(memory_space=pl.ANY),
            pl.BlockSpec((BR, D), lambda i, b: (i, 0)),
            pl.BlockSpec(memory_space=pltpu.SMEM),
        ],
        out_specs=pl.BlockSpec((1, BR, D), lambda i, b: (b, i, 0)),
        out_shape=jax.ShapeDtypeStruct((B, MAX_LEN, D), jnp.float32),
        input_output_aliases={0: 0},
    )
    return tc_run(out_sc, tshift, len_i32)
